# baseline (speedup 1.0000x reference)
import jax
import jax.numpy as jnp
from jax.experimental import pallas as pl

_N = 10000
_B = 4
_IN = 10
_HID = 64
_K = 20
_H = 1


_TR = 256
_CT = 79
_NP = _CT * 128
_BIG = 3.0e38
_BIGI = 1 << 30
_TOPC = 4


def _knn_kernel(rows_ref, emb3_ref, sq3_ref, idx_ref):
    i = pl.program_id(0)
    rows = rows_ref[...]
    rowg = i * _TR + jax.lax.broadcasted_iota(jnp.int32, (_TR, 128), 0)
    lane = jax.lax.broadcasted_iota(jnp.int32, (_TR, 128), 1)

    def body(t, carry):
        ms = list(carry[:_TOPC])
        ts = list(carry[_TOPC:])
        e = emb3_ref[t]
        dt = jnp.dot(rows, e, preferred_element_type=jnp.float32)
        dt = dt + sq3_ref[t, 0:1, :]
        colg = t * 128 + lane
        bad = (colg == rowg) | (colg >= _N)
        v = jnp.where(bad, _BIG, dt)
        ti = jnp.full((_TR, 128), t, jnp.int32)
        for p in range(_TOPC):
            c = v < ms[p]
            nm = jnp.where(c, v, ms[p])
            nt = jnp.where(c, ti, ts[p])
            v, ti = jnp.where(c, ms[p], v), jnp.where(c, ts[p], ti)
            ms[p], ts[p] = nm, nt
        return tuple(ms) + tuple(ts)

    init = tuple(jnp.full((_TR, 128), _BIG) for _ in range(_TOPC)) + \
           tuple(jnp.zeros((_TR, 128), jnp.int32) for _ in range(_TOPC))
    carry = jax.lax.fori_loop(0, _CT, body, init)
    vals = jnp.concatenate(carry[:_TOPC], axis=1)
    gcol = jnp.concatenate([carry[_TOPC + p] * 128 + lane for p in range(_TOPC)],
                           axis=1)
    sels = []
    for _ in range(_K):
        mn = jnp.min(vals, axis=1, keepdims=True)
        eq = vals == mn
        sel = jnp.min(jnp.where(eq, gcol, _BIGI), axis=1, keepdims=True)
        sels.append(sel)
        vals = jnp.where(eq & (gcol == sel), _BIG, vals)
    idx_ref[...] = jnp.concatenate(sels, axis=1)


def _knn_edges(emb):
    sq = jnp.sum(emb * emb, axis=1)
    rows2 = emb * -2.0
    embT = jnp.pad(emb.T, ((0, 0), (0, _NP - _N)))
    emb3 = embT.reshape(64, _CT, 128).transpose(1, 0, 2)
    sqp = jnp.pad(sq, (0, _NP - _N)).reshape(_CT, 1, 128)
    sq3 = jnp.pad(sqp, ((0, 0), (0, 7), (0, 0)))
    return pl.pallas_call(
        _knn_kernel,
        grid=(_N // _TR + (1 if _N % _TR else 0),),
        in_specs=[
            pl.BlockSpec((_TR, 64), lambda i: (i, 0)),
            pl.BlockSpec((_CT, 64, 128), lambda i: (0, 0, 0)),
            pl.BlockSpec((_CT, 8, 128), lambda i: (0, 0, 0)),
        ],
        out_specs=pl.BlockSpec((_TR, _K), lambda i: (i, 0)),
        out_shape=jax.ShapeDtypeStruct((_N // _TR * _TR + (_TR if _N % _TR else 0), _K),
                                       jnp.int32),
    )(jnp.pad(rows2, ((0, (-_N) % _TR), (0, 0))), emb3, sq3)[:_N]


def _bn(v, g, b, eps=1e-5):
    mean = v.mean(axis=0)
    var = v.var(axis=0)
    return (v - mean) / jnp.sqrt(var + eps) * g + b


def _head_kernel(o_ref, w_ref, b_ref, out_ref):
    out_ref[...] = o_ref[...] @ w_ref[...] + b_ref[0, 0]


def kernel(batch_x, emb_table, lin_W, lin_b, att_src, att_dst, gat_bias,
           bn1_gamma, bn1_beta, bn2_gamma, bn2_beta, out_W, out_b):
    M = _B * _N
    x = batch_x.reshape(-1, _IN)
    idx = _knn_edges(emb_table)
    emb_rep = jnp.tile(emb_table, (_B, 1))
    xh = (x @ lin_W + lin_b)
    a_src_x = att_src[0, 0, :_HID]
    a_src_e = att_src[0, 0, _HID:]
    a_dst_x = att_dst[0, 0, :_HID]
    a_dst_e = att_dst[0, 0, _HID:]
    a_src = xh @ a_src_x + emb_rep @ a_src_e
    a_dst = xh @ a_dst_x + emb_rep @ a_dst_e

    nbr = jnp.concatenate([idx, jnp.arange(_N)[:, None]], axis=1)
    offs = (jnp.arange(_B) * _N)[:, None, None]
    nbrB = (nbr[None] + offs).reshape(M, _K + 1)

    alpha = a_src[nbrB] + a_dst[:, None]
    alpha = jax.nn.leaky_relu(alpha, negative_slope=0.2)
    amax = alpha.max(axis=1, keepdims=True)
    ae = jnp.exp(alpha - amax)
    den = ae.sum(axis=1, keepdims=True)
    w = ae / (den + 1e-16)
    msgs = xh[nbrB]
    out = jnp.einsum("mk,mkh->mh", w, msgs)
    out = out + gat_bias
    out = _bn(out, bn1_gamma, bn1_beta)
    out = jax.nn.relu(out)
    out = out * emb_rep
    out = _bn(out, bn2_gamma, bn2_beta)
    out = pl.pallas_call(
        _head_kernel,
        out_shape=jax.ShapeDtypeStruct((M, 1), jnp.float32),
    )(out, out_W, out_b.reshape(1, 1))
    return out.reshape(_B, _N)
